# baseline (speedup 1.0000x reference)
import functools

import jax
import jax.numpy as jnp
from jax import lax
from jax.experimental import pallas as pl
from jax.experimental.pallas import tpu as pltpu
from jax.experimental.pallas import tpu_sc as plsc

NC = 2
NS = 16
NW = NC * NS

G = 128
GROUPS = 4
CHUNK = G * GROUPS


def _gather_kernel(hidden, n_chunks_w, blocks_per_seq, idx_hbm, table_hbm,
                   out_hbm, idx0, idx1, rows0, rows1, gsem0, gsem1, wsem0,
                   wsem1):
    wid = lax.axis_index("s") * NC + lax.axis_index("c")
    chunk_base = wid * n_chunks_w

    bufs = ((idx0, rows0, gsem0, wsem0), (idx1, rows1, gsem1, wsem1))

    def dst_ref(c):
        cg = chunk_base + c
        s = cg // blocks_per_seq
        j = cg % blocks_per_seq
        return out_hbm.at[s, pl.ds(j * CHUNK, CHUNK)]

    def issue(c, sub):
        idx_v, rows_v, gsem, _ = bufs[sub]
        pltpu.sync_copy(idx_hbm.at[pl.ds((chunk_base + c) * CHUNK, CHUNK)],
                        idx_v)
        pltpu.async_copy(table_hbm.at[idx_v], rows_v, gsem)

    def drain(c, sub):
        idx_v, rows_v, gsem, wsem = bufs[sub]
        pltpu.make_async_copy(table_hbm.at[idx_v], rows_v, gsem).wait()
        dst = dst_ref(c)
        pltpu.async_copy(rows_v, dst, wsem)
        pltpu.make_async_copy(rows_v, dst, wsem).wait()

    issue(0, 0)
    issue(1, 1)

    def body(k, _):
        drain(2 * k, 0)
        issue(2 * k + 2, 0)
        drain(2 * k + 1, 1)

        @pl.when(2 * k + 3 < n_chunks_w)
        def _():
            issue(2 * k + 3, 1)

        return ()

    lax.fori_loop(0, (n_chunks_w - 1) // 2, body, (), unroll=False)
    drain(n_chunks_w - 1, 0)


def kernel(input_ids, position_ids, word_embeddings):
    batch, seq = input_ids.shape
    vocab, hidden = word_embeddings.shape
    n = batch * seq
    assert batch % CHUNK == 0
    blocks_per_seq = batch // CHUNK
    n_chunks = n // CHUNK
    assert (n_chunks // 2) % NW == 0

    idx_flat = input_ids.T.reshape(n)

    half_seq = seq // 2
    half_chunks_w = (n_chunks // 2) // NW
    assert half_chunks_w % 2 == 1 and half_chunks_w >= 3

    mesh = plsc.VectorSubcoreMesh(core_axis_name="c", subcore_axis_name="s")
    gather = pl.kernel(
        functools.partial(_gather_kernel, hidden, half_chunks_w,
                          blocks_per_seq),
        out_type=jax.ShapeDtypeStruct((half_seq, batch, hidden), jnp.float32),
        mesh=mesh,
        scratch_types=[
            pltpu.VMEM((CHUNK,), jnp.int32),
            pltpu.VMEM((CHUNK,), jnp.int32),
            pltpu.VMEM((CHUNK, hidden), jnp.float32),
            pltpu.VMEM((CHUNK, hidden), jnp.float32),
            pltpu.SemaphoreType.DMA,
            pltpu.SemaphoreType.DMA,
            pltpu.SemaphoreType.DMA,
            pltpu.SemaphoreType.DMA,
        ],
        compiler_params=pltpu.CompilerParams(use_tc_tiling_on_sc=False),
    )
    out0 = gather(idx_flat[: n // 2], word_embeddings)
    out1 = gather(idx_flat[n // 2:], word_embeddings)
    out = jnp.concatenate([out0, out1], axis=0)
    return (out.transpose(1, 0, 2), position_ids)


# speedup vs baseline: 1.0399x; 1.0399x over previous
import functools

import jax
import jax.numpy as jnp
from jax import lax
from jax.experimental import pallas as pl
from jax.experimental.pallas import tpu as pltpu
from jax.experimental.pallas import tpu_sc as plsc

NC = 2
NS = 16
NW = NC * NS

G = 128
GROUPS = 10
CHUNK = G * GROUPS


def _gather_kernel(hidden, n_pairs, idx_hbm, table_hbm, out_hbm,
                   idx0, idx1, rows0, rows1, gsem0, gsem1, wsem0, wsem1):
    wid = lax.axis_index("s") * NC + lax.axis_index("c")
    n_chunks = 2 * n_pairs
    row_base = wid * (n_chunks * CHUNK)

    bufs = ((idx0, rows0, gsem0, wsem0), (idx1, rows1, gsem1, wsem1))

    def issue(c, sub):
        idx_v, rows_v, gsem, _ = bufs[sub]
        pltpu.sync_copy(idx_hbm.at[pl.ds(row_base + c * CHUNK, CHUNK)],
                        idx_v)
        pltpu.async_copy(table_hbm.at[idx_v], rows_v, gsem)

    def drain(c, sub):
        idx_v, rows_v, gsem, wsem = bufs[sub]
        pltpu.make_async_copy(table_hbm.at[idx_v], rows_v, gsem).wait()
        dst = out_hbm.at[pl.ds(row_base + c * CHUNK, CHUNK)]
        pltpu.async_copy(rows_v, dst, wsem)
        pltpu.make_async_copy(rows_v, dst, wsem).wait()

    issue(0, 0)
    issue(1, 1)

    def body(k, _):
        drain(2 * k, 0)
        issue(2 * k + 2, 0)
        drain(2 * k + 1, 1)
        issue(2 * k + 3, 1)
        return ()

    lax.fori_loop(0, n_pairs - 1, body, (), unroll=False)

    drain(n_chunks - 2, 0)
    drain(n_chunks - 1, 1)


def kernel(input_ids, position_ids, word_embeddings):
    batch, seq = input_ids.shape
    vocab, hidden = word_embeddings.shape
    n = batch * seq
    assert n % (NW * 2 * CHUNK) == 0
    n_pairs = n // (NW * 2 * CHUNK)

    idx_flat = input_ids.T.reshape(n)

    mesh = plsc.VectorSubcoreMesh(core_axis_name="c", subcore_axis_name="s")
    gather = pl.kernel(
        functools.partial(_gather_kernel, hidden, n_pairs),
        out_type=jax.ShapeDtypeStruct((n, hidden), jnp.float32),
        mesh=mesh,
        scratch_types=[
            pltpu.VMEM((CHUNK,), jnp.int32),
            pltpu.VMEM((CHUNK,), jnp.int32),
            pltpu.VMEM((CHUNK, hidden), jnp.float32),
            pltpu.VMEM((CHUNK, hidden), jnp.float32),
            pltpu.SemaphoreType.DMA,
            pltpu.SemaphoreType.DMA,
            pltpu.SemaphoreType.DMA,
            pltpu.SemaphoreType.DMA,
        ],
        compiler_params=pltpu.CompilerParams(use_tc_tiling_on_sc=False),
    )
    out = gather(idx_flat, word_embeddings)
    return (out.reshape(seq, batch, hidden).transpose(1, 0, 2), position_ids)
